# Initial kernel scaffold; baseline (speedup 1.0000x reference)
import jax
import jax.numpy as jnp
from jax.experimental import pallas as pl


def kernel(x1, edge_index1, edge_attr1, batch1, x2, edge_index2, edge_attr2, batch2, W_i1, W_h1, W_o1, W_i2, W_h2, W_o2, ffn_W1, ffn_b1, ffn_W2, ffn_b2, ffn_W3, ffn_b3):
    raise NotImplementedError("write your pallas kernel here")


# speedup vs baseline: 1.4663x; 1.4663x over previous
import jax
import jax.numpy as jnp
from jax import lax
from jax.experimental import pallas as pl
from jax.experimental.pallas import tpu as pltpu
from jax.experimental.pallas import tpu_sc as plsc

N = 10000
E = 320000
EH = E // 2
D_FEAT = 128
D_EDGE = 16
HID = 128
B = 128
FFN_HID = 256

_NC, _NS = 2, 16
_NW = _NC * _NS
_CH = 80
_EPT = E // _NW
_CPT = _EPT // _CH

_RE = 2000
_NBE = E // _RE
_NBH = EH // _RE
_RN = 2000
_NBN = N // _RN

_mesh = plsc.VectorSubcoreMesh(
    core_axis_name="c", subcore_axis_name="s", num_cores=_NC, num_subcores=_NS)

_f32 = jnp.float32


def _sc_gather_body(table, idx, out, idxb, rowb, sem):
    w = lax.axis_index("c") * _NS + lax.axis_index("s")
    pltpu.sync_copy(idx.at[w], idxb)
    out0 = w * _EPT

    def chunk(j, carry):
        pltpu.async_copy(table.at[idxb.at[j]], rowb, sem).wait()
        pltpu.sync_copy(rowb, out.at[pl.ds(out0 + j * _CH, _CH)])
        return carry

    lax.fori_loop(0, _CPT, chunk, 0)


def _sc_gather(table, idx3):
    fn = pl.kernel(
        _sc_gather_body,
        out_type=jax.ShapeDtypeStruct((E, HID), _f32),
        mesh=_mesh,
        scratch_types=[
            pltpu.VMEM((_CPT, _CH), jnp.int32),
            pltpu.VMEM((_CH, HID), _f32),
            pltpu.SemaphoreType.DMA,
        ],
    )
    return fn(table, idx3)


def _sc_segsum_body(rows, idx, zeros, out, idxb, rowb, acc, sem):
    c = lax.axis_index("c")
    s = lax.axis_index("s")
    w = c * _NS + s

    @pl.when(s < _NS - 1)
    def _():
        pltpu.sync_copy(zeros.at[pl.ds(s * 624, 624)], acc.at[pl.ds(s * 624, 624)])

    @pl.when(s == _NS - 1)
    def _():
        pltpu.sync_copy(zeros.at[pl.ds(624 * 15, 640)], acc.at[pl.ds(624 * 15, 640)])

    pltpu.sync_copy(idx.at[w], idxb)
    plsc.subcore_barrier()

    in0 = w * _EPT

    def chunk(j, carry):
        pltpu.sync_copy(rows.at[pl.ds(in0 + j * _CH, _CH)], rowb)
        pltpu.sync_copy(rowb, acc.at[idxb.at[j]], add=True)
        return carry

    lax.fori_loop(0, _CPT, chunk, 0)
    plsc.subcore_barrier()

    @pl.when(s < _NS - 1)
    def _():
        pltpu.sync_copy(acc.at[pl.ds(s * 624, 624)],
                        out.at[c, pl.ds(s * 624, 624)])

    @pl.when(s == _NS - 1)
    def _():
        pltpu.sync_copy(acc.at[pl.ds(624 * 15, 640)],
                        out.at[c, pl.ds(624 * 15, 640)])


def _sc_segsum(rows, idx3, zeros):
    fn = pl.kernel(
        _sc_segsum_body,
        out_type=jax.ShapeDtypeStruct((_NC, N, HID), _f32),
        mesh=_mesh,
        scratch_types=[
            pltpu.VMEM((_CPT, _CH), jnp.int32),
            pltpu.VMEM((_CH, HID), _f32),
            pltpu.VMEM_SHARED((N, HID), _f32),
            pltpu.SemaphoreType.DMA,
        ],
    )
    return fn(rows, idx3, zeros)


def _nodepre_body(x_ref, w_ref, out_ref):
    out_ref[...] = jnp.dot(x_ref[...], w_ref[...], preferred_element_type=_f32)


def _tc_nodepre(x, w_ix):
    return pl.pallas_call(
        _nodepre_body,
        grid=(_NBN,),
        in_specs=[
            pl.BlockSpec((_RN, D_FEAT), lambda i: (i, 0)),
            pl.BlockSpec((D_FEAT, HID), lambda i: (0, 0)),
        ],
        out_specs=pl.BlockSpec((_RN, HID), lambda i: (i, 0)),
        out_shape=jax.ShapeDtypeStruct((N, HID), _f32),
    )(x, w_ix)


def _merge_body(p_ref, out_ref):
    out_ref[...] = p_ref[0] + p_ref[1]


def _tc_merge(p):
    return pl.pallas_call(
        _merge_body,
        grid=(_NBN,),
        in_specs=[pl.BlockSpec((_NC, _RN, HID), lambda i: (0, i, 0))],
        out_specs=pl.BlockSpec((_RN, HID), lambda i: (i, 0)),
        out_shape=jax.ShapeDtypeStruct((N, HID), _f32),
    )(p)


def _edge0_body(ea_ref, xs_ref, wie_ref, wh_ref, h0_ref, hw_ref):
    h0 = jnp.maximum(
        xs_ref[...] + jnp.dot(ea_ref[...], wie_ref[...],
                              preferred_element_type=_f32), 0.0)
    h0_ref[...] = h0
    hw_ref[...] = jnp.dot(h0, wh_ref[...], preferred_element_type=_f32)


def _tc_edge0(ea_di, xwsrc, w_ie, w_h):
    return pl.pallas_call(
        _edge0_body,
        grid=(_NBE,),
        in_specs=[
            pl.BlockSpec((_RE, D_EDGE), lambda i: (i, 0)),
            pl.BlockSpec((_RE, HID), lambda i: (i, 0)),
            pl.BlockSpec((D_EDGE, HID), lambda i: (0, 0)),
            pl.BlockSpec((HID, HID), lambda i: (0, 0)),
        ],
        out_specs=[
            pl.BlockSpec((_RE, HID), lambda i: (i, 0)),
            pl.BlockSpec((_RE, HID), lambda i: (i, 0)),
        ],
        out_shape=[
            jax.ShapeDtypeStruct((E, HID), _f32),
            jax.ShapeDtypeStruct((E, HID), _f32),
        ],
    )(ea_di, xwsrc, w_ie, w_h)


def _step_body(h0_ref, gs_ref, hwsw_ref, wh_ref, out_ref):
    h = jnp.maximum(h0_ref[...] + gs_ref[...] - hwsw_ref[...], 0.0)
    out_ref[...] = jnp.dot(h, wh_ref[...], preferred_element_type=_f32)


def _tc_step(h0, gs, hw, w_h):
    return pl.pallas_call(
        _step_body,
        grid=(_NBE,),
        in_specs=[
            pl.BlockSpec((_RE, HID), lambda i: (i, 0)),
            pl.BlockSpec((_RE, HID), lambda i: (i, 0)),
            pl.BlockSpec((_RE, HID), lambda i: ((i + _NBH) % _NBE, 0)),
            pl.BlockSpec((HID, HID), lambda i: (0, 0)),
        ],
        out_specs=pl.BlockSpec((_RE, HID), lambda i: (i, 0)),
        out_shape=jax.ShapeDtypeStruct((E, HID), _f32),
    )(h0, gs, hw, w_h)


def _last_body(h0_ref, gs_ref, hwsw_ref, out_ref):
    out_ref[...] = jnp.maximum(h0_ref[...] + gs_ref[...] - hwsw_ref[...], 0.0)


def _tc_last(h0, gs, hw):
    return pl.pallas_call(
        _last_body,
        grid=(_NBE,),
        in_specs=[
            pl.BlockSpec((_RE, HID), lambda i: (i, 0)),
            pl.BlockSpec((_RE, HID), lambda i: (i, 0)),
            pl.BlockSpec((_RE, HID), lambda i: ((i + _NBH) % _NBE, 0)),
        ],
        out_specs=pl.BlockSpec((_RE, HID), lambda i: (i, 0)),
        out_shape=jax.ShapeDtypeStruct((E, HID), _f32),
    )(h0, gs, hw)


def _nodeout_body(x_ref, nm_ref, wox_ref, wom_ref, out_ref):
    nm = nm_ref[0] + nm_ref[1]
    hv = jnp.dot(x_ref[...], wox_ref[...], preferred_element_type=_f32)
    hv = hv + jnp.dot(nm, wom_ref[...], preferred_element_type=_f32)
    out_ref[...] = jnp.maximum(hv, 0.0)


def _tc_nodeout(x, nm, w_ox, w_om):
    return pl.pallas_call(
        _nodeout_body,
        grid=(_NBN,),
        in_specs=[
            pl.BlockSpec((_RN, D_FEAT), lambda i: (i, 0)),
            pl.BlockSpec((_NC, _RN, HID), lambda i: (0, i, 0)),
            pl.BlockSpec((D_FEAT, HID), lambda i: (0, 0)),
            pl.BlockSpec((HID, HID), lambda i: (0, 0)),
        ],
        out_specs=pl.BlockSpec((_RN, HID), lambda i: (i, 0)),
        out_shape=jax.ShapeDtypeStruct((N, HID), _f32),
    )(x, nm, w_ox, w_om)


def _pool_body(hv_ref, b_ref, sums_ref, cnt_ref):
    i = pl.program_id(0)

    @pl.when(i == 0)
    def _():
        sums_ref[...] = jnp.zeros_like(sums_ref)
        cnt_ref[...] = jnp.zeros_like(cnt_ref)

    ids = b_ref[0, 0, :]
    oneh = (ids[:, None] == lax.broadcasted_iota(jnp.int32, (_RN, B), 1)).astype(_f32)
    dn = (((0,), (0,)), ((), ()))
    sums_ref[...] += lax.dot_general(oneh, hv_ref[...], dn,
                                     preferred_element_type=_f32)
    cnt_ref[...] += lax.dot_general(oneh, jnp.ones((_RN, HID), _f32), dn,
                                    preferred_element_type=_f32)


def _tc_pool(hv, batch3):
    return pl.pallas_call(
        _pool_body,
        grid=(_NBN,),
        in_specs=[
            pl.BlockSpec((_RN, HID), lambda i: (i, 0)),
            pl.BlockSpec((1, 1, _RN), lambda i: (i, 0, 0)),
        ],
        out_specs=[
            pl.BlockSpec((B, HID), lambda i: (0, 0)),
            pl.BlockSpec((B, HID), lambda i: (0, 0)),
        ],
        out_shape=[
            jax.ShapeDtypeStruct((B, HID), _f32),
            jax.ShapeDtypeStruct((B, HID), _f32),
        ],
    )(hv, batch3)


def _ffn_body(s1, c1, s2, c2, w1, b1, w2, b2, w3, b3, out):
    v1 = s1[...] / jnp.maximum(c1[...], 1.0)
    v2 = s2[...] / jnp.maximum(c2[...], 1.0)
    v = jnp.concatenate([v1, v2], axis=1)
    h = jnp.maximum(jnp.dot(v, w1[...], preferred_element_type=_f32) + b1[...], 0.0)
    h = jnp.maximum(jnp.dot(h, w2[...], preferred_element_type=_f32) + b2[...], 0.0)
    out[...] = jnp.dot(h, w3[...], preferred_element_type=_f32) + b3[...]


def _tc_ffn(s1, c1, s2, c2, fw1, fb1, fw2, fb2, fw3, fb3):
    def full(shape):
        return pl.BlockSpec(shape, lambda: tuple(0 for _ in shape))
    return pl.pallas_call(
        _ffn_body,
        in_specs=[
            full((B, HID)), full((B, HID)), full((B, HID)), full((B, HID)),
            full((2 * HID, FFN_HID)), full((1, FFN_HID)),
            full((FFN_HID, FFN_HID)), full((1, FFN_HID)),
            full((FFN_HID, 1)), full((1, 1)),
        ],
        out_specs=full((B, 1)),
        out_shape=jax.ShapeDtypeStruct((B, 1), _f32),
    )(s1, c1, s2, c2, fw1, fb1.reshape(1, -1), fw2, fb2.reshape(1, -1),
      fw3, fb3.reshape(1, 1))


def _mpn(x, ei, ea, batch, w_i, w_h, w_o, zeros):
    s = ei[0, 0::2]
    d = ei[1, 0::2]
    idxg3 = jnp.concatenate([s, d]).reshape(_NW, _CPT, _CH)
    idxs3 = jnp.concatenate([d, s]).reshape(_NW, _CPT, _CH)
    ea_di = jnp.concatenate([ea[0::2], ea[1::2]])

    w_ix, w_ie = w_i[:D_FEAT], w_i[D_FEAT:]
    w_ox, w_om = w_o[:D_FEAT], w_o[D_FEAT:]

    xw = _tc_nodepre(x, w_ix)
    xwsrc = _sc_gather(xw, idxg3)
    h0, hw = _tc_edge0(ea_di, xwsrc, w_ie, w_h)

    p = _sc_segsum(hw, idxs3, zeros)
    gs = _sc_gather(_tc_merge(p), idxg3)
    hw = _tc_step(h0, gs, hw, w_h)

    p = _sc_segsum(hw, idxs3, zeros)
    gs = _sc_gather(_tc_merge(p), idxg3)
    h2 = _tc_last(h0, gs, hw)

    p = _sc_segsum(h2, idxs3, zeros)
    hv = _tc_nodeout(x, p, w_ox, w_om)
    return _tc_pool(hv, batch.reshape(_NBN, 1, _RN))


def kernel(x1, edge_index1, edge_attr1, batch1,
           x2, edge_index2, edge_attr2, batch2,
           W_i1, W_h1, W_o1, W_i2, W_h2, W_o2,
           ffn_W1, ffn_b1, ffn_W2, ffn_b2, ffn_W3, ffn_b3):
    zeros = jnp.zeros((N, HID), _f32)
    s1, c1 = _mpn(x1, edge_index1, edge_attr1, batch1, W_i1, W_h1, W_o1, zeros)
    s2, c2 = _mpn(x2, edge_index2, edge_attr2, batch2, W_i2, W_h2, W_o2, zeros)
    return _tc_ffn(s1, c1, s2, c2, ffn_W1, ffn_b1, ffn_W2, ffn_b2, ffn_W3, ffn_b3)


# speedup vs baseline: 1.5943x; 1.0873x over previous
import jax
import jax.numpy as jnp
from jax import lax
from jax.experimental import pallas as pl
from jax.experimental.pallas import tpu as pltpu
from jax.experimental.pallas import tpu_sc as plsc

N = 10000
E = 320000
EH = E // 2
D_FEAT = 128
D_EDGE = 16
HID = 128
B = 128
FFN_HID = 256

_NC, _NS = 2, 16
_NW = _NC * _NS
_CH = 80
_EPT = E // _NW
_CPT = _EPT // _CH

_RE = 2000
_NBE = E // _RE
_NBH = EH // _RE
_RN = 2000
_NBN = N // _RN

_mesh = plsc.VectorSubcoreMesh(
    core_axis_name="c", subcore_axis_name="s", num_cores=_NC, num_subcores=_NS)

_f32 = jnp.float32


_GK = 5
_NG = _CPT // _GK
_GR = _GK * _CH


def _sc_gather_body(table, idx, out, idxb, buf_a, buf_b, semg, semw_a, semw_b):
    w = lax.axis_index("c") * _NS + lax.axis_index("s")
    pltpu.sync_copy(idx.at[w], idxb)
    out0 = w * _EPT

    def group(i, buf, semw, first):
        @pl.when(jnp.logical_not(first))
        def _():
            pltpu.make_async_copy(buf, out.at[pl.ds(out0, _GR)], semw).wait()
        descs = [
            pltpu.async_copy(table.at[idxb.at[i * _GK + b]],
                             buf.at[pl.ds(b * _CH, _CH)], semg)
            for b in range(_GK)
        ]
        for dsc in descs:
            dsc.wait()
        pltpu.async_copy(buf, out.at[pl.ds(out0 + i * _GR, _GR)], semw)

    def body(i, carry):
        even = (i % 2) == 0

        @pl.when(even)
        def _():
            group(i, buf_a, semw_a, i == 0)

        @pl.when(jnp.logical_not(even))
        def _():
            group(i, buf_b, semw_b, i == 1)

        return carry

    lax.fori_loop(0, _NG, body, 0)
    pltpu.make_async_copy(buf_a, out.at[pl.ds(out0, _GR)], semw_a).wait()
    pltpu.make_async_copy(buf_b, out.at[pl.ds(out0, _GR)], semw_b).wait()


def _sc_gather(table, idx3):
    fn = pl.kernel(
        _sc_gather_body,
        out_type=jax.ShapeDtypeStruct((E, HID), _f32),
        mesh=_mesh,
        scratch_types=[
            pltpu.VMEM((_CPT, _CH), jnp.int32),
            pltpu.VMEM((_GR, HID), _f32),
            pltpu.VMEM((_GR, HID), _f32),
            pltpu.SemaphoreType.DMA,
            pltpu.SemaphoreType.DMA,
            pltpu.SemaphoreType.DMA,
        ],
    )
    return fn(table, idx3)


def _sc_segsum_body(rows, idx, zeros, out, idxb, buf_a, buf_b, acc,
                    semr_a, semr_b, semsc):
    c = lax.axis_index("c")
    s = lax.axis_index("s")
    w = c * _NS + s

    @pl.when(s < _NS - 1)
    def _():
        pltpu.sync_copy(zeros.at[pl.ds(s * 624, 624)], acc.at[pl.ds(s * 624, 624)])

    @pl.when(s == _NS - 1)
    def _():
        pltpu.sync_copy(zeros.at[pl.ds(624 * 15, 640)], acc.at[pl.ds(624 * 15, 640)])

    pltpu.sync_copy(idx.at[w], idxb)
    plsc.subcore_barrier()

    in0 = w * _EPT
    pltpu.async_copy(rows.at[pl.ds(in0, _CH)], buf_a, semr_a)

    def chunk(i, buf, semr, nbuf, nsemr):
        pltpu.make_async_copy(rows.at[pl.ds(in0, _CH)], buf, semr).wait()

        @pl.when(i > 0)
        def _():
            pltpu.make_async_copy(nbuf, acc.at[idxb.at[i]], semsc).wait()

        @pl.when(i + 1 < _CPT)
        def _():
            pltpu.async_copy(rows.at[pl.ds(in0 + (i + 1) * _CH, _CH)], nbuf, nsemr)

        pltpu.async_copy(buf, acc.at[idxb.at[i]], semsc, add=True)

    def body(i, carry):
        even = (i % 2) == 0

        @pl.when(even)
        def _():
            chunk(i, buf_a, semr_a, buf_b, semr_b)

        @pl.when(jnp.logical_not(even))
        def _():
            chunk(i, buf_b, semr_b, buf_a, semr_a)

        return carry

    lax.fori_loop(0, _CPT, body, 0)
    pltpu.make_async_copy(buf_a, acc.at[idxb.at[_CPT - 1]], semsc).wait()
    plsc.subcore_barrier()

    @pl.when(s < _NS - 1)
    def _():
        pltpu.sync_copy(acc.at[pl.ds(s * 624, 624)],
                        out.at[c, pl.ds(s * 624, 624)])

    @pl.when(s == _NS - 1)
    def _():
        pltpu.sync_copy(acc.at[pl.ds(624 * 15, 640)],
                        out.at[c, pl.ds(624 * 15, 640)])


def _sc_segsum(rows, idx3, zeros):
    fn = pl.kernel(
        _sc_segsum_body,
        out_type=jax.ShapeDtypeStruct((_NC, N, HID), _f32),
        mesh=_mesh,
        scratch_types=[
            pltpu.VMEM((_CPT, _CH), jnp.int32),
            pltpu.VMEM((_CH, HID), _f32),
            pltpu.VMEM((_CH, HID), _f32),
            pltpu.VMEM_SHARED((N, HID), _f32),
            pltpu.SemaphoreType.DMA,
            pltpu.SemaphoreType.DMA,
            pltpu.SemaphoreType.DMA,
        ],
    )
    return fn(rows, idx3, zeros)


def _nodepre_body(x_ref, w_ref, out_ref):
    out_ref[...] = jnp.dot(x_ref[...], w_ref[...], preferred_element_type=_f32)


def _tc_nodepre(x, w_ix):
    return pl.pallas_call(
        _nodepre_body,
        grid=(_NBN,),
        in_specs=[
            pl.BlockSpec((_RN, D_FEAT), lambda i: (i, 0)),
            pl.BlockSpec((D_FEAT, HID), lambda i: (0, 0)),
        ],
        out_specs=pl.BlockSpec((_RN, HID), lambda i: (i, 0)),
        out_shape=jax.ShapeDtypeStruct((N, HID), _f32),
    )(x, w_ix)


def _merge_body(p_ref, out_ref):
    out_ref[...] = p_ref[0] + p_ref[1]


def _tc_merge(p):
    return pl.pallas_call(
        _merge_body,
        grid=(_NBN,),
        in_specs=[pl.BlockSpec((_NC, _RN, HID), lambda i: (0, i, 0))],
        out_specs=pl.BlockSpec((_RN, HID), lambda i: (i, 0)),
        out_shape=jax.ShapeDtypeStruct((N, HID), _f32),
    )(p)


def _edge0_body(ea_ref, xs_ref, wie_ref, wh_ref, h0_ref, hw_ref):
    h0 = jnp.maximum(
        xs_ref[...] + jnp.dot(ea_ref[...], wie_ref[...],
                              preferred_element_type=_f32), 0.0)
    h0_ref[...] = h0
    hw_ref[...] = jnp.dot(h0, wh_ref[...], preferred_element_type=_f32)


def _tc_edge0(ea_di, xwsrc, w_ie, w_h):
    return pl.pallas_call(
        _edge0_body,
        grid=(_NBE,),
        in_specs=[
            pl.BlockSpec((_RE, D_EDGE), lambda i: (i, 0)),
            pl.BlockSpec((_RE, HID), lambda i: (i, 0)),
            pl.BlockSpec((D_EDGE, HID), lambda i: (0, 0)),
            pl.BlockSpec((HID, HID), lambda i: (0, 0)),
        ],
        out_specs=[
            pl.BlockSpec((_RE, HID), lambda i: (i, 0)),
            pl.BlockSpec((_RE, HID), lambda i: (i, 0)),
        ],
        out_shape=[
            jax.ShapeDtypeStruct((E, HID), _f32),
            jax.ShapeDtypeStruct((E, HID), _f32),
        ],
    )(ea_di, xwsrc, w_ie, w_h)


def _step_body(h0_ref, gs_ref, hwsw_ref, wh_ref, out_ref):
    h = jnp.maximum(h0_ref[...] + gs_ref[...] - hwsw_ref[...], 0.0)
    out_ref[...] = jnp.dot(h, wh_ref[...], preferred_element_type=_f32)


def _tc_step(h0, gs, hw, w_h):
    return pl.pallas_call(
        _step_body,
        grid=(_NBE,),
        in_specs=[
            pl.BlockSpec((_RE, HID), lambda i: (i, 0)),
            pl.BlockSpec((_RE, HID), lambda i: (i, 0)),
            pl.BlockSpec((_RE, HID), lambda i: ((i + _NBH) % _NBE, 0)),
            pl.BlockSpec((HID, HID), lambda i: (0, 0)),
        ],
        out_specs=pl.BlockSpec((_RE, HID), lambda i: (i, 0)),
        out_shape=jax.ShapeDtypeStruct((E, HID), _f32),
    )(h0, gs, hw, w_h)


def _last_body(h0_ref, gs_ref, hwsw_ref, out_ref):
    out_ref[...] = jnp.maximum(h0_ref[...] + gs_ref[...] - hwsw_ref[...], 0.0)


def _tc_last(h0, gs, hw):
    return pl.pallas_call(
        _last_body,
        grid=(_NBE,),
        in_specs=[
            pl.BlockSpec((_RE, HID), lambda i: (i, 0)),
            pl.BlockSpec((_RE, HID), lambda i: (i, 0)),
            pl.BlockSpec((_RE, HID), lambda i: ((i + _NBH) % _NBE, 0)),
        ],
        out_specs=pl.BlockSpec((_RE, HID), lambda i: (i, 0)),
        out_shape=jax.ShapeDtypeStruct((E, HID), _f32),
    )(h0, gs, hw)


def _nodeout_body(x_ref, nm_ref, wox_ref, wom_ref, out_ref):
    nm = nm_ref[0] + nm_ref[1]
    hv = jnp.dot(x_ref[...], wox_ref[...], preferred_element_type=_f32)
    hv = hv + jnp.dot(nm, wom_ref[...], preferred_element_type=_f32)
    out_ref[...] = jnp.maximum(hv, 0.0)


def _tc_nodeout(x, nm, w_ox, w_om):
    return pl.pallas_call(
        _nodeout_body,
        grid=(_NBN,),
        in_specs=[
            pl.BlockSpec((_RN, D_FEAT), lambda i: (i, 0)),
            pl.BlockSpec((_NC, _RN, HID), lambda i: (0, i, 0)),
            pl.BlockSpec((D_FEAT, HID), lambda i: (0, 0)),
            pl.BlockSpec((HID, HID), lambda i: (0, 0)),
        ],
        out_specs=pl.BlockSpec((_RN, HID), lambda i: (i, 0)),
        out_shape=jax.ShapeDtypeStruct((N, HID), _f32),
    )(x, nm, w_ox, w_om)


def _pool_body(hv_ref, b_ref, sums_ref, cnt_ref):
    i = pl.program_id(0)

    @pl.when(i == 0)
    def _():
        sums_ref[...] = jnp.zeros_like(sums_ref)
        cnt_ref[...] = jnp.zeros_like(cnt_ref)

    ids = b_ref[0, 0, :]
    oneh = (ids[:, None] == lax.broadcasted_iota(jnp.int32, (_RN, B), 1)).astype(_f32)
    dn = (((0,), (0,)), ((), ()))
    sums_ref[...] += lax.dot_general(oneh, hv_ref[...], dn,
                                     preferred_element_type=_f32)
    cnt_ref[...] += lax.dot_general(oneh, jnp.ones((_RN, HID), _f32), dn,
                                    preferred_element_type=_f32)


def _tc_pool(hv, batch3):
    return pl.pallas_call(
        _pool_body,
        grid=(_NBN,),
        in_specs=[
            pl.BlockSpec((_RN, HID), lambda i: (i, 0)),
            pl.BlockSpec((1, 1, _RN), lambda i: (i, 0, 0)),
        ],
        out_specs=[
            pl.BlockSpec((B, HID), lambda i: (0, 0)),
            pl.BlockSpec((B, HID), lambda i: (0, 0)),
        ],
        out_shape=[
            jax.ShapeDtypeStruct((B, HID), _f32),
            jax.ShapeDtypeStruct((B, HID), _f32),
        ],
    )(hv, batch3)


def _ffn_body(s1, c1, s2, c2, w1, b1, w2, b2, w3, b3, out):
    v1 = s1[...] / jnp.maximum(c1[...], 1.0)
    v2 = s2[...] / jnp.maximum(c2[...], 1.0)
    v = jnp.concatenate([v1, v2], axis=1)
    h = jnp.maximum(jnp.dot(v, w1[...], preferred_element_type=_f32) + b1[...], 0.0)
    h = jnp.maximum(jnp.dot(h, w2[...], preferred_element_type=_f32) + b2[...], 0.0)
    out[...] = jnp.dot(h, w3[...], preferred_element_type=_f32) + b3[...]


def _tc_ffn(s1, c1, s2, c2, fw1, fb1, fw2, fb2, fw3, fb3):
    def full(shape):
        return pl.BlockSpec(shape, lambda: tuple(0 for _ in shape))
    return pl.pallas_call(
        _ffn_body,
        in_specs=[
            full((B, HID)), full((B, HID)), full((B, HID)), full((B, HID)),
            full((2 * HID, FFN_HID)), full((1, FFN_HID)),
            full((FFN_HID, FFN_HID)), full((1, FFN_HID)),
            full((FFN_HID, 1)), full((1, 1)),
        ],
        out_specs=full((B, 1)),
        out_shape=jax.ShapeDtypeStruct((B, 1), _f32),
    )(s1, c1, s2, c2, fw1, fb1.reshape(1, -1), fw2, fb2.reshape(1, -1),
      fw3, fb3.reshape(1, 1))


def _mpn(x, ei, ea, batch, w_i, w_h, w_o, zeros):
    s = ei[0, 0::2]
    d = ei[1, 0::2]
    idxg3 = jnp.concatenate([s, d]).reshape(_NW, _CPT, _CH)
    idxs3 = jnp.concatenate([d, s]).reshape(_NW, _CPT, _CH)
    ea_di = jnp.concatenate([ea[0::2], ea[1::2]])

    w_ix, w_ie = w_i[:D_FEAT], w_i[D_FEAT:]
    w_ox, w_om = w_o[:D_FEAT], w_o[D_FEAT:]

    xw = _tc_nodepre(x, w_ix)
    xwsrc = _sc_gather(xw, idxg3)
    h0, hw = _tc_edge0(ea_di, xwsrc, w_ie, w_h)

    p = _sc_segsum(hw, idxs3, zeros)
    gs = _sc_gather(_tc_merge(p), idxg3)
    hw = _tc_step(h0, gs, hw, w_h)

    p = _sc_segsum(hw, idxs3, zeros)
    gs = _sc_gather(_tc_merge(p), idxg3)
    h2 = _tc_last(h0, gs, hw)

    p = _sc_segsum(h2, idxs3, zeros)
    hv = _tc_nodeout(x, p, w_ox, w_om)
    return _tc_pool(hv, batch.reshape(_NBN, 1, _RN))


def kernel(x1, edge_index1, edge_attr1, batch1,
           x2, edge_index2, edge_attr2, batch2,
           W_i1, W_h1, W_o1, W_i2, W_h2, W_o2,
           ffn_W1, ffn_b1, ffn_W2, ffn_b2, ffn_W3, ffn_b3):
    zeros = jnp.zeros((N, HID), _f32)
    s1, c1 = _mpn(x1, edge_index1, edge_attr1, batch1, W_i1, W_h1, W_o1, zeros)
    s2, c2 = _mpn(x2, edge_index2, edge_attr2, batch2, W_i2, W_h2, W_o2, zeros)
    return _tc_ffn(s1, c1, s2, c2, ffn_W1, ffn_b1, ffn_W2, ffn_b2, ffn_W3, ffn_b3)


# speedup vs baseline: 1.6795x; 1.0534x over previous
import jax
import jax.numpy as jnp
from jax import lax
from jax.experimental import pallas as pl
from jax.experimental.pallas import tpu as pltpu
from jax.experimental.pallas import tpu_sc as plsc

N = 10000
E = 320000
EH = E // 2
D_FEAT = 128
D_EDGE = 16
HID = 128
B = 128
FFN_HID = 256

_NC, _NS = 2, 16
_NW = _NC * _NS
_CH = 80
_EPT = E // _NW
_CPT = _EPT // _CH

_RE = 2000
_NBE = E // _RE
_NBH = EH // _RE
_RN = 2000
_NBN = N // _RN

_mesh = plsc.VectorSubcoreMesh(
    core_axis_name="c", subcore_axis_name="s", num_cores=_NC, num_subcores=_NS)

_f32 = jnp.float32


_GK = 5
_NG = _CPT // _GK
_GR = _GK * _CH


def _sc_gather_body(table, idx, out, idxb, buf_a, buf_b, tspm,
                    semg_a, semg_b, semw):
    s = lax.axis_index("s")
    w = lax.axis_index("c") * _NS + s
    pltpu.sync_copy(idx.at[w], idxb)

    @pl.when(s < _NS - 1)
    def _():
        pltpu.sync_copy(table.at[pl.ds(s * 624, 624)], tspm.at[pl.ds(s * 624, 624)])

    @pl.when(s == _NS - 1)
    def _():
        pltpu.sync_copy(table.at[pl.ds(624 * 15, 640)], tspm.at[pl.ds(624 * 15, 640)])

    plsc.subcore_barrier()
    out0 = w * _EPT
    pltpu.async_copy(tspm.at[idxb.at[0]], buf_a, semg_a)

    def chunk(i, buf, semg, nbuf, nsemg):
        pltpu.make_async_copy(tspm.at[idxb.at[i]], buf, semg).wait()

        @pl.when(i > 0)
        def _():
            pltpu.make_async_copy(nbuf, out.at[pl.ds(out0, _CH)], semw).wait()

        @pl.when(i + 1 < _CPT)
        def _():
            pltpu.async_copy(tspm.at[idxb.at[i + 1]], nbuf, nsemg)

        pltpu.async_copy(buf, out.at[pl.ds(out0 + i * _CH, _CH)], semw)

    def body(i, carry):
        even = (i % 2) == 0

        @pl.when(even)
        def _():
            chunk(i, buf_a, semg_a, buf_b, semg_b)

        @pl.when(jnp.logical_not(even))
        def _():
            chunk(i, buf_b, semg_b, buf_a, semg_a)

        return carry

    lax.fori_loop(0, _CPT, body, 0)
    pltpu.make_async_copy(buf_a, out.at[pl.ds(out0, _CH)], semw).wait()


def _sc_gather(table, idx3):
    fn = pl.kernel(
        _sc_gather_body,
        out_type=jax.ShapeDtypeStruct((E, HID), _f32),
        mesh=_mesh,
        scratch_types=[
            pltpu.VMEM((_CPT, _CH), jnp.int32),
            pltpu.VMEM((_CH, HID), _f32),
            pltpu.VMEM((_CH, HID), _f32),
            pltpu.VMEM_SHARED((N, HID), _f32),
            pltpu.SemaphoreType.DMA,
            pltpu.SemaphoreType.DMA,
            pltpu.SemaphoreType.DMA,
        ],
    )
    return fn(table, idx3)


def _sc_segsum_body(rows, idx, zeros, out, idxb, buf_a, buf_b, acc,
                    semr_a, semr_b, semsc):
    c = lax.axis_index("c")
    s = lax.axis_index("s")
    w = c * _NS + s

    @pl.when(s < _NS - 1)
    def _():
        pltpu.sync_copy(zeros.at[pl.ds(s * 624, 624)], acc.at[pl.ds(s * 624, 624)])

    @pl.when(s == _NS - 1)
    def _():
        pltpu.sync_copy(zeros.at[pl.ds(624 * 15, 640)], acc.at[pl.ds(624 * 15, 640)])

    pltpu.sync_copy(idx.at[w], idxb)
    plsc.subcore_barrier()

    in0 = w * _EPT
    pltpu.async_copy(rows.at[pl.ds(in0, _CH)], buf_a, semr_a)

    def chunk(i, buf, semr, nbuf, nsemr):
        pltpu.make_async_copy(rows.at[pl.ds(in0, _CH)], buf, semr).wait()

        @pl.when(i > 0)
        def _():
            pltpu.make_async_copy(nbuf, acc.at[idxb.at[i]], semsc).wait()

        @pl.when(i + 1 < _CPT)
        def _():
            pltpu.async_copy(rows.at[pl.ds(in0 + (i + 1) * _CH, _CH)], nbuf, nsemr)

        pltpu.async_copy(buf, acc.at[idxb.at[i]], semsc, add=True)

    def body(i, carry):
        even = (i % 2) == 0

        @pl.when(even)
        def _():
            chunk(i, buf_a, semr_a, buf_b, semr_b)

        @pl.when(jnp.logical_not(even))
        def _():
            chunk(i, buf_b, semr_b, buf_a, semr_a)

        return carry

    lax.fori_loop(0, _CPT, body, 0)
    pltpu.make_async_copy(buf_a, acc.at[idxb.at[_CPT - 1]], semsc).wait()
    plsc.subcore_barrier()

    @pl.when(s < _NS - 1)
    def _():
        pltpu.sync_copy(acc.at[pl.ds(s * 624, 624)],
                        out.at[c, pl.ds(s * 624, 624)])

    @pl.when(s == _NS - 1)
    def _():
        pltpu.sync_copy(acc.at[pl.ds(624 * 15, 640)],
                        out.at[c, pl.ds(624 * 15, 640)])


def _sc_segsum(rows, idx3, zeros):
    fn = pl.kernel(
        _sc_segsum_body,
        out_type=jax.ShapeDtypeStruct((_NC, N, HID), _f32),
        mesh=_mesh,
        scratch_types=[
            pltpu.VMEM((_CPT, _CH), jnp.int32),
            pltpu.VMEM((_CH, HID), _f32),
            pltpu.VMEM((_CH, HID), _f32),
            pltpu.VMEM_SHARED((N, HID), _f32),
            pltpu.SemaphoreType.DMA,
            pltpu.SemaphoreType.DMA,
            pltpu.SemaphoreType.DMA,
        ],
    )
    return fn(rows, idx3, zeros)


def _nodepre_body(x_ref, w_ref, out_ref):
    out_ref[...] = jnp.dot(x_ref[...], w_ref[...], preferred_element_type=_f32)


def _tc_nodepre(x, w_ix):
    return pl.pallas_call(
        _nodepre_body,
        grid=(_NBN,),
        in_specs=[
            pl.BlockSpec((_RN, D_FEAT), lambda i: (i, 0)),
            pl.BlockSpec((D_FEAT, HID), lambda i: (0, 0)),
        ],
        out_specs=pl.BlockSpec((_RN, HID), lambda i: (i, 0)),
        out_shape=jax.ShapeDtypeStruct((N, HID), _f32),
    )(x, w_ix)


def _merge_body(p_ref, out_ref):
    out_ref[...] = p_ref[0] + p_ref[1]


def _tc_merge(p):
    return pl.pallas_call(
        _merge_body,
        grid=(_NBN,),
        in_specs=[pl.BlockSpec((_NC, _RN, HID), lambda i: (0, i, 0))],
        out_specs=pl.BlockSpec((_RN, HID), lambda i: (i, 0)),
        out_shape=jax.ShapeDtypeStruct((N, HID), _f32),
    )(p)


def _edge0_body(ea_ref, xs_ref, wie_ref, wh_ref, h0_ref, hw_ref):
    h0 = jnp.maximum(
        xs_ref[...] + jnp.dot(ea_ref[...], wie_ref[...],
                              preferred_element_type=_f32), 0.0)
    h0_ref[...] = h0
    hw_ref[...] = jnp.dot(h0, wh_ref[...], preferred_element_type=_f32)


def _tc_edge0(ea_di, xwsrc, w_ie, w_h):
    return pl.pallas_call(
        _edge0_body,
        grid=(_NBE,),
        in_specs=[
            pl.BlockSpec((_RE, D_EDGE), lambda i: (i, 0)),
            pl.BlockSpec((_RE, HID), lambda i: (i, 0)),
            pl.BlockSpec((D_EDGE, HID), lambda i: (0, 0)),
            pl.BlockSpec((HID, HID), lambda i: (0, 0)),
        ],
        out_specs=[
            pl.BlockSpec((_RE, HID), lambda i: (i, 0)),
            pl.BlockSpec((_RE, HID), lambda i: (i, 0)),
        ],
        out_shape=[
            jax.ShapeDtypeStruct((E, HID), _f32),
            jax.ShapeDtypeStruct((E, HID), _f32),
        ],
    )(ea_di, xwsrc, w_ie, w_h)


def _step_body(h0_ref, gs_ref, hwsw_ref, wh_ref, out_ref):
    h = jnp.maximum(h0_ref[...] + gs_ref[...] - hwsw_ref[...], 0.0)
    out_ref[...] = jnp.dot(h, wh_ref[...], preferred_element_type=_f32)


def _tc_step(h0, gs, hw, w_h):
    return pl.pallas_call(
        _step_body,
        grid=(_NBE,),
        in_specs=[
            pl.BlockSpec((_RE, HID), lambda i: (i, 0)),
            pl.BlockSpec((_RE, HID), lambda i: (i, 0)),
            pl.BlockSpec((_RE, HID), lambda i: ((i + _NBH) % _NBE, 0)),
            pl.BlockSpec((HID, HID), lambda i: (0, 0)),
        ],
        out_specs=pl.BlockSpec((_RE, HID), lambda i: (i, 0)),
        out_shape=jax.ShapeDtypeStruct((E, HID), _f32),
    )(h0, gs, hw, w_h)


def _last_body(h0_ref, gs_ref, hwsw_ref, out_ref):
    out_ref[...] = jnp.maximum(h0_ref[...] + gs_ref[...] - hwsw_ref[...], 0.0)


def _tc_last(h0, gs, hw):
    return pl.pallas_call(
        _last_body,
        grid=(_NBE,),
        in_specs=[
            pl.BlockSpec((_RE, HID), lambda i: (i, 0)),
            pl.BlockSpec((_RE, HID), lambda i: (i, 0)),
            pl.BlockSpec((_RE, HID), lambda i: ((i + _NBH) % _NBE, 0)),
        ],
        out_specs=pl.BlockSpec((_RE, HID), lambda i: (i, 0)),
        out_shape=jax.ShapeDtypeStruct((E, HID), _f32),
    )(h0, gs, hw)


def _nodeout_body(x_ref, nm_ref, wox_ref, wom_ref, out_ref):
    nm = nm_ref[0] + nm_ref[1]
    hv = jnp.dot(x_ref[...], wox_ref[...], preferred_element_type=_f32)
    hv = hv + jnp.dot(nm, wom_ref[...], preferred_element_type=_f32)
    out_ref[...] = jnp.maximum(hv, 0.0)


def _tc_nodeout(x, nm, w_ox, w_om):
    return pl.pallas_call(
        _nodeout_body,
        grid=(_NBN,),
        in_specs=[
            pl.BlockSpec((_RN, D_FEAT), lambda i: (i, 0)),
            pl.BlockSpec((_NC, _RN, HID), lambda i: (0, i, 0)),
            pl.BlockSpec((D_FEAT, HID), lambda i: (0, 0)),
            pl.BlockSpec((HID, HID), lambda i: (0, 0)),
        ],
        out_specs=pl.BlockSpec((_RN, HID), lambda i: (i, 0)),
        out_shape=jax.ShapeDtypeStruct((N, HID), _f32),
    )(x, nm, w_ox, w_om)


def _pool_body(hv_ref, b_ref, sums_ref, cnt_ref):
    i = pl.program_id(0)

    @pl.when(i == 0)
    def _():
        sums_ref[...] = jnp.zeros_like(sums_ref)
        cnt_ref[...] = jnp.zeros_like(cnt_ref)

    ids = b_ref[0, 0, :]
    oneh = (ids[:, None] == lax.broadcasted_iota(jnp.int32, (_RN, B), 1)).astype(_f32)
    dn = (((0,), (0,)), ((), ()))
    sums_ref[...] += lax.dot_general(oneh, hv_ref[...], dn,
                                     preferred_element_type=_f32)
    cnt_ref[...] += lax.dot_general(oneh, jnp.ones((_RN, HID), _f32), dn,
                                    preferred_element_type=_f32)


def _tc_pool(hv, batch3):
    return pl.pallas_call(
        _pool_body,
        grid=(_NBN,),
        in_specs=[
            pl.BlockSpec((_RN, HID), lambda i: (i, 0)),
            pl.BlockSpec((1, 1, _RN), lambda i: (i, 0, 0)),
        ],
        out_specs=[
            pl.BlockSpec((B, HID), lambda i: (0, 0)),
            pl.BlockSpec((B, HID), lambda i: (0, 0)),
        ],
        out_shape=[
            jax.ShapeDtypeStruct((B, HID), _f32),
            jax.ShapeDtypeStruct((B, HID), _f32),
        ],
    )(hv, batch3)


def _ffn_body(s1, c1, s2, c2, w1, b1, w2, b2, w3, b3, out):
    v1 = s1[...] / jnp.maximum(c1[...], 1.0)
    v2 = s2[...] / jnp.maximum(c2[...], 1.0)
    v = jnp.concatenate([v1, v2], axis=1)
    h = jnp.maximum(jnp.dot(v, w1[...], preferred_element_type=_f32) + b1[...], 0.0)
    h = jnp.maximum(jnp.dot(h, w2[...], preferred_element_type=_f32) + b2[...], 0.0)
    out[...] = jnp.dot(h, w3[...], preferred_element_type=_f32) + b3[...]


def _tc_ffn(s1, c1, s2, c2, fw1, fb1, fw2, fb2, fw3, fb3):
    def full(shape):
        return pl.BlockSpec(shape, lambda: tuple(0 for _ in shape))
    return pl.pallas_call(
        _ffn_body,
        in_specs=[
            full((B, HID)), full((B, HID)), full((B, HID)), full((B, HID)),
            full((2 * HID, FFN_HID)), full((1, FFN_HID)),
            full((FFN_HID, FFN_HID)), full((1, FFN_HID)),
            full((FFN_HID, 1)), full((1, 1)),
        ],
        out_specs=full((B, 1)),
        out_shape=jax.ShapeDtypeStruct((B, 1), _f32),
    )(s1, c1, s2, c2, fw1, fb1.reshape(1, -1), fw2, fb2.reshape(1, -1),
      fw3, fb3.reshape(1, 1))


def _mpn(x, ei, ea, batch, w_i, w_h, w_o, zeros):
    s = ei[0, 0::2]
    d = ei[1, 0::2]
    idxg3 = jnp.concatenate([s, d]).reshape(_NW, _CPT, _CH)
    idxs3 = jnp.concatenate([d, s]).reshape(_NW, _CPT, _CH)
    ea_di = jnp.concatenate([ea[0::2], ea[1::2]])

    w_ix, w_ie = w_i[:D_FEAT], w_i[D_FEAT:]
    w_ox, w_om = w_o[:D_FEAT], w_o[D_FEAT:]

    xw = _tc_nodepre(x, w_ix)
    xwsrc = _sc_gather(xw, idxg3)
    h0, hw = _tc_edge0(ea_di, xwsrc, w_ie, w_h)

    p = _sc_segsum(hw, idxs3, zeros)
    gs = _sc_gather(_tc_merge(p), idxg3)
    hw = _tc_step(h0, gs, hw, w_h)

    p = _sc_segsum(hw, idxs3, zeros)
    gs = _sc_gather(_tc_merge(p), idxg3)
    h2 = _tc_last(h0, gs, hw)

    p = _sc_segsum(h2, idxs3, zeros)
    hv = _tc_nodeout(x, p, w_ox, w_om)
    return _tc_pool(hv, batch.reshape(_NBN, 1, _RN))


def kernel(x1, edge_index1, edge_attr1, batch1,
           x2, edge_index2, edge_attr2, batch2,
           W_i1, W_h1, W_o1, W_i2, W_h2, W_o2,
           ffn_W1, ffn_b1, ffn_W2, ffn_b2, ffn_W3, ffn_b3):
    zeros = jnp.zeros((N, HID), _f32)
    s1, c1 = _mpn(x1, edge_index1, edge_attr1, batch1, W_i1, W_h1, W_o1, zeros)
    s2, c2 = _mpn(x2, edge_index2, edge_attr2, batch2, W_i2, W_h2, W_o2, zeros)
    return _tc_ffn(s1, c1, s2, c2, ffn_W1, ffn_b1, ffn_W2, ffn_b2, ffn_W3, ffn_b3)
